# baseline (speedup 1.0000x reference)
import functools

import jax
import jax.numpy as jnp
from jax import lax
from jax.experimental import pallas as pl
from jax.experimental.pallas import tpu as pltpu
from jax.experimental.pallas import tpu_sc as plsc

_TOTAL = 32768
_D = 128
_NB = 16
_LAM = 1e-05
_NCORE = 1
_NSUB = 16
_CHUNK = _TOTAL * 4 // _NSUB


def _hist_body(ids_hbm, out_hbm, buf, cnt, sem):
    c = lax.axis_index("c")
    s = lax.axis_index("s")
    wid = s * _NCORE + c
    cp = pltpu.make_async_copy(ids_hbm.at[pl.ds(wid * _CHUNK, _CHUNK)], buf, sem)
    cp.start()

    zv = jnp.zeros((16,), jnp.int32)
    for k in range(_NB):
        cnt[pl.ds(k * 16, 16)] = zv
    cp.wait()

    lo = buf[pl.ds(0, 16)][0]
    hi = buf[pl.ds(_CHUNK - 16, 16)][12]

    def per_bin(b, carry):
        def body(k, acc):
            for j in range(4):
                v = buf[pl.ds(k * 64 + j * 16, 16)]
                acc = acc + jnp.where(v == b, jnp.int32(1), jnp.int32(0))
            return acc

        acc = lax.fori_loop(0, _CHUNK // 64, body, jnp.zeros((16,), jnp.int32))
        cnt[pl.ds(b * 16, 16)] = acc
        return carry

    lax.fori_loop(lo, hi + 1, per_bin, 0)
    pltpu.sync_copy(cnt, out_hbm.at[wid])


@functools.cache
def _hist():
    return pl.kernel(
        _hist_body,
        mesh=plsc.VectorSubcoreMesh(core_axis_name="c", subcore_axis_name="s", num_cores=1),
        out_type=jax.ShapeDtypeStruct((_NSUB, _NB * 16), jnp.int32),
        scratch_types=[
            pltpu.VMEM((_CHUNK,), jnp.int32),
            pltpu.VMEM((_NB * 16,), jnp.int32),
            pltpu.SemaphoreType.DMA,
        ],
    )


def _dense_body(f_ref, part_ref, o_ref, *, tile):
    f = f_ref[...]
    p = part_ref[...].astype(jnp.float32)
    lanemask = lax.broadcasted_iota(jnp.int32, (1, 16), 1) % 4 == 0
    prows = jnp.sum(jnp.where(lanemask, p, 0.0), axis=1, keepdims=True)
    rid = lax.broadcasted_iota(jnp.int32, (_NSUB * _NB, 1), 0) % _NB
    sel = rid == lax.broadcasted_iota(jnp.int32, (1, _NB), 1)
    counts = jnp.sum(jnp.where(sel, prows, 0.0), axis=0, keepdims=True)

    inc = counts
    for sh in (1, 2, 4, 8):
        inc = inc + jnp.concatenate(
            [jnp.zeros((1, sh), jnp.float32), inc[:, : _NB - sh]], axis=1
        )
    starts = inc - counts

    row0 = pl.program_id(0) * tile
    gid = (row0 + lax.broadcasted_iota(jnp.int32, (tile, 1), 0)).astype(
        jnp.float32
    )
    inb = ((gid >= starts) & (gid < inc)).astype(jnp.float32)
    n = lax.dot_general(
        inb, counts, (((1,), (1,)), ((), ())),
        preferred_element_type=jnp.float32,
    )

    ones = jnp.ones((_D, 1), jnp.float32)
    dn = (((1,), (0,)), ((), ()))
    s1 = lax.dot_general(f, ones, dn, preferred_element_type=jnp.float32)
    mean = s1 * (1.0 / _D)
    d = f - mean
    sq = d * d
    rs = lax.dot_general(sq, ones, dn, preferred_element_type=jnp.float32)
    nm1 = n - 1.0
    r2 = (0.125 * nm1) / (rs + _LAM * nm1)
    t = sq * r2 + 0.25
    o_ref[...] = f * (1.5 + 0.5 * jnp.tanh(t))


def kernel(features, indices):
    ids_flat = indices.reshape(-1)
    partials = _hist()(ids_flat).reshape(_NSUB * _NB, 16)
    tile = 4096
    out = pl.pallas_call(
        functools.partial(_dense_body, tile=tile),
        grid=(_TOTAL // tile,),
        in_specs=[
            pl.BlockSpec((tile, _D), lambda i: (i, 0)),
            pl.BlockSpec((_NSUB * _NB, 16), lambda i: (0, 0)),
        ],
        out_specs=pl.BlockSpec((tile, _D), lambda i: (i, 0)),
        out_shape=jax.ShapeDtypeStruct((_TOTAL, _D), jnp.float32),
    )(features, partials)
    return out


# speedup vs baseline: 1.0174x; 1.0174x over previous
import functools

import jax
import jax.numpy as jnp
from jax import lax
from jax.experimental import pallas as pl
from jax.experimental.pallas import tpu as pltpu
from jax.experimental.pallas import tpu_sc as plsc

_TOTAL = 32768
_D = 128
_NB = 16
_LAM = 1e-05
_NCORE = 1
_NSUB = 16
_CHUNK = _TOTAL * 4 // _NSUB


def _hist_body(ids_hbm, out_hbm, buf, cnt, sem):
    c = lax.axis_index("c")
    s = lax.axis_index("s")
    wid = s * _NCORE + c
    cp = pltpu.make_async_copy(ids_hbm.at[pl.ds(wid * _CHUNK, _CHUNK)], buf, sem)
    cp.start()

    zv = jnp.zeros((16,), jnp.int32)
    for k in range(_NB):
        cnt[pl.ds(k * 16, 16)] = zv
    cp.wait()

    lo = buf[pl.ds(0, 16)][0]
    hi = buf[pl.ds(_CHUNK - 16, 16)][12]

    def per_bin(b, carry):
        def body(k, acc):
            for j in range(4):
                v = buf[pl.ds(k * 64 + j * 16, 16)]
                acc = acc + jnp.where(v == b, jnp.int32(1), jnp.int32(0))
            return acc

        acc = lax.fori_loop(0, _CHUNK // 64, body, jnp.zeros((16,), jnp.int32))
        cnt[pl.ds(b * 16, 16)] = acc
        return carry

    lax.fori_loop(lo, hi + 1, per_bin, 0)
    pltpu.sync_copy(cnt, out_hbm.at[wid])


@functools.cache
def _hist():
    return pl.kernel(
        _hist_body,
        mesh=plsc.VectorSubcoreMesh(core_axis_name="c", subcore_axis_name="s", num_cores=1),
        out_type=jax.ShapeDtypeStruct((_NSUB, _NB * 16), jnp.int32),
        scratch_types=[
            pltpu.VMEM((_CHUNK,), jnp.int32),
            pltpu.VMEM((_NB * 16,), jnp.int32),
            pltpu.SemaphoreType.DMA,
        ],
    )


def _dense_body(f_ref, part_ref, o_ref, *, tile):
    f = f_ref[...]
    p = part_ref[...].astype(jnp.float32)
    lanemask = lax.broadcasted_iota(jnp.int32, (1, 16), 1) % 4 == 0
    prows = jnp.sum(jnp.where(lanemask, p, 0.0), axis=1, keepdims=True)
    rid = lax.broadcasted_iota(jnp.int32, (_NSUB * _NB, 1), 0) % _NB
    sel = rid == lax.broadcasted_iota(jnp.int32, (1, _NB), 1)
    counts = jnp.sum(jnp.where(sel, prows, 0.0), axis=0, keepdims=True)

    inc = counts
    for sh in (1, 2, 4, 8):
        inc = inc + jnp.concatenate(
            [jnp.zeros((1, sh), jnp.float32), inc[:, : _NB - sh]], axis=1
        )
    starts = inc - counts

    row0 = pl.program_id(0) * tile
    gid = (row0 + lax.broadcasted_iota(jnp.int32, (tile, 1), 0)).astype(
        jnp.float32
    )
    inb = ((gid >= starts) & (gid < inc)).astype(jnp.float32)
    n = lax.dot_general(
        inb, counts, (((1,), (1,)), ((), ())),
        preferred_element_type=jnp.float32,
    )

    ones = jnp.ones((_D, 1), jnp.float32)
    dn = (((1,), (0,)), ((), ()))
    s1 = lax.dot_general(f, ones, dn, preferred_element_type=jnp.float32)
    mean = s1 * (1.0 / _D)
    d = f - mean
    sq = d * d
    rs = lax.dot_general(sq, ones, dn, preferred_element_type=jnp.float32)
    nm1 = n - 1.0
    r2 = (0.125 * nm1) / (rs + _LAM * nm1)
    t = sq * r2 + 0.25
    o_ref[...] = f * (1.5 + 0.5 * jnp.tanh(t))


def kernel(features, indices):
    ids_flat = indices.reshape(-1)
    partials = _hist()(ids_flat).reshape(_NSUB * _NB, 16)
    tile = 8192
    out = pl.pallas_call(
        functools.partial(_dense_body, tile=tile),
        grid=(_TOTAL // tile,),
        in_specs=[
            pl.BlockSpec((tile, _D), lambda i: (i, 0)),
            pl.BlockSpec((_NSUB * _NB, 16), lambda i: (0, 0)),
        ],
        out_specs=pl.BlockSpec((tile, _D), lambda i: (i, 0)),
        out_shape=jax.ShapeDtypeStruct((_TOTAL, _D), jnp.float32),
    )(features, partials)
    return out


# speedup vs baseline: 1.0634x; 1.0452x over previous
import functools

import jax
import jax.numpy as jnp
from jax import lax
from jax.experimental import pallas as pl
from jax.experimental.pallas import tpu as pltpu
from jax.experimental.pallas import tpu_sc as plsc

_TOTAL = 32768
_D = 128
_NB = 16
_LAM = 1e-05
_NCORE = 1
_NSUB = 16
_CHUNK = _TOTAL * 4 // _NSUB


def _hist_body(ids_hbm, out_hbm, buf, cnt, sem):
    c = lax.axis_index("c")
    s = lax.axis_index("s")
    wid = s * _NCORE + c
    cp = pltpu.make_async_copy(ids_hbm.at[pl.ds(wid * _CHUNK, _CHUNK)], buf, sem)
    cp.start()

    zv = jnp.zeros((16,), jnp.int32)
    for k in range(_NB):
        cnt[pl.ds(k * 16, 16)] = zv
    cp.wait()

    lo = buf[pl.ds(0, 16)][0]
    hi = buf[pl.ds(_CHUNK - 16, 16)][12]

    def per_bin(b, carry):
        def body(k, acc):
            for j in range(4):
                v = buf[pl.ds(k * 64 + j * 16, 16)]
                acc = acc + jnp.where(v == b, jnp.int32(1), jnp.int32(0))
            return acc

        acc = lax.fori_loop(0, _CHUNK // 64, body, jnp.zeros((16,), jnp.int32))
        cnt[pl.ds(b * 16, 16)] = acc
        return carry

    lax.fori_loop(lo, hi + 1, per_bin, 0)
    pltpu.sync_copy(cnt, out_hbm.at[wid])


@functools.cache
def _hist():
    return pl.kernel(
        _hist_body,
        mesh=plsc.VectorSubcoreMesh(core_axis_name="c", subcore_axis_name="s", num_cores=1),
        out_type=jax.ShapeDtypeStruct((_NSUB, _NB * 16), jnp.int32),
        scratch_types=[
            pltpu.VMEM((_CHUNK,), jnp.int32),
            pltpu.VMEM((_NB * 16,), jnp.int32),
            pltpu.SemaphoreType.DMA,
        ],
    )


def _dense_body(f_ref, part_ref, o_ref, *, tile):
    f = f_ref[...]
    p = part_ref[...].astype(jnp.float32)
    lanemask = lax.broadcasted_iota(jnp.int32, (1, 16), 1) % 4 == 0
    prows = jnp.sum(jnp.where(lanemask, p, 0.0), axis=1, keepdims=True)
    rid = lax.broadcasted_iota(jnp.int32, (_NSUB * _NB, 1), 0) % _NB
    sel = rid == lax.broadcasted_iota(jnp.int32, (1, _NB), 1)
    counts = jnp.sum(jnp.where(sel, prows, 0.0), axis=0, keepdims=True)

    inc = counts
    for sh in (1, 2, 4, 8):
        inc = inc + jnp.concatenate(
            [jnp.zeros((1, sh), jnp.float32), inc[:, : _NB - sh]], axis=1
        )
    starts = inc - counts

    row0 = pl.program_id(0) * tile
    gid = (row0 + lax.broadcasted_iota(jnp.int32, (tile, 1), 0)).astype(
        jnp.float32
    )
    inb = ((gid >= starts) & (gid < inc)).astype(jnp.float32)
    dn = (((1,), (0,)), ((), ()))
    n = lax.dot_general(
        inb * counts, jnp.ones((_NB, _D), jnp.float32), dn,
        preferred_element_type=jnp.float32,
    )

    ones_bb = jnp.ones((_D, _D), jnp.float32)
    mean = lax.dot_general(f, ones_bb, dn, preferred_element_type=jnp.float32) * (
        1.0 / _D
    )
    d = f - mean
    sq = d * d
    rs = lax.dot_general(sq, ones_bb, dn, preferred_element_type=jnp.float32)
    nm1 = n - 1.0
    r2 = (0.125 * nm1) / (rs + _LAM * nm1)
    t = sq * r2 + 0.25
    o_ref[...] = f * (1.5 + 0.5 * jnp.tanh(t))


def kernel(features, indices):
    ids_flat = indices.reshape(-1)
    partials = _hist()(ids_flat).reshape(_NSUB * _NB, 16)
    tile = 8192
    out = pl.pallas_call(
        functools.partial(_dense_body, tile=tile),
        grid=(_TOTAL // tile,),
        in_specs=[
            pl.BlockSpec((tile, _D), lambda i: (i, 0)),
            pl.BlockSpec((_NSUB * _NB, 16), lambda i: (0, 0)),
        ],
        out_specs=pl.BlockSpec((tile, _D), lambda i: (i, 0)),
        out_shape=jax.ShapeDtypeStruct((_TOTAL, _D), jnp.float32),
    )(features, partials)
    return out
